# baseline (speedup 1.0000x reference)
import functools

import jax
import jax.numpy as jnp
from jax import lax
from jax.experimental import pallas as pl
from jax.experimental.pallas import tpu as pltpu
from jax.experimental.pallas import tpu_sc as plsc

N = 10000
D = 128
E = 320000
NC = 2
NS = 16
NW = NC * NS
EW = E // NW
L = 16
K = 16
SEGE = 2000
NSEG = EW // SEGE
NBS = SEGE // K
R = 8
G = 6
NBS2 = 128
SEG2 = NBS2 * K
NSEG2 = 6
EWP = NSEG2 * SEG2
NPAD = 10112
RPT = NPAD // NS


def _dense_body(x_ref, wv_ref, wr_ref, bv_ref, aw_ref, ab_ref,
                v_ref, p_ref, q_ref):
    x = x_ref[...]
    v_ref[...] = lax.dot_general(
        x, wv_ref[...], (((1,), (1,)), ((), ())),
        precision=lax.Precision.HIGHEST,
        preferred_element_type=jnp.float32) + bv_ref[...][None, :]
    a1 = aw_ref[0:D][None, :]
    a2 = aw_ref[D:2 * D][None, :]
    w2 = lax.dot_general(a2, wr_ref[...], (((1,), (0,)), ((), ())),
                         precision=lax.Precision.HIGHEST,
                         preferred_element_type=jnp.float32)
    p_ref[...] = lax.dot_general(a1, x, (((1,), (1,)), ((), ())),
                                 precision=lax.Precision.HIGHEST,
                                 preferred_element_type=jnp.float32) + ab_ref[0]
    q_ref[...] = lax.dot_general(w2, x, (((1,), (1,)), ((), ())),
                                 precision=lax.Precision.HIGHEST,
                                 preferred_element_type=jnp.float32)


_dense = pl.pallas_call(
    _dense_body,
    out_shape=(jax.ShapeDtypeStruct((N, D), jnp.float32),
               jax.ShapeDtypeStruct((1, N), jnp.float32),
               jax.ShapeDtypeStruct((1, N), jnp.float32)),
    in_specs=[pl.BlockSpec(memory_space=pltpu.VMEM)] * 5
             + [pl.BlockSpec(memory_space=pltpu.SMEM)],
)


_mesh = plsc.VectorSubcoreMesh(core_axis_name="c", subcore_axis_name="s")


@functools.partial(
    pl.kernel,
    mesh=_mesh,
    compiler_params=pltpu.CompilerParams(needs_layout_passes=False),
    out_type=(jax.ShapeDtypeStruct((NW * EWP,), jnp.int32),
              jax.ShapeDtypeStruct((NW * EWP,), jnp.int32),
              jax.ShapeDtypeStruct((NW * EWP,), jnp.float32),
              jax.ShapeDtypeStruct((NW * 16,), jnp.int32),
              jax.ShapeDtypeStruct((NW, 1, N), jnp.float32)),
    scratch_types=[
        pltpu.VMEM((1, N), jnp.float32),
        pltpu.VMEM((1, N), jnp.float32),
        pltpu.VMEM((SEGE,), jnp.int32),
        pltpu.VMEM((SEGE,), jnp.int32),
        pltpu.VMEM((SEGE,), jnp.float32),
        pltpu.VMEM((1, N), jnp.float32),
        pltpu.VMEM((EWP,), jnp.int32),
        pltpu.VMEM((EWP,), jnp.int32),
        pltpu.VMEM((EWP,), jnp.float32),
        pltpu.VMEM((16,), jnp.int32),
    ],
)
def _edge_w(p_hbm, q_hbm, src_hbm, dst_hbm, lab_hbm,
            cs_hbm, cd_hbm, cw_hbm, cnt_hbm, den_hbm,
            p_v, q_v, s_v, d_v, l_v, den_v, cs_v, cd_v, cw_v, cnt_v):
    cid = lax.axis_index("c")
    sid = lax.axis_index("s")
    wid = sid * NC + cid
    base = wid * EW

    pltpu.sync_copy(p_hbm, p_v)
    pltpu.sync_copy(q_hbm, q_v)

    def zero_body(i, _):
        den_v[0, pl.ds(i * L, L)] = jnp.zeros((L,), jnp.float32)
        return 0
    lax.fori_loop(0, N // L, zero_body, 0)

    def seg_body(g, off):
        seg_base = base + g * SEGE
        pltpu.sync_copy(src_hbm.at[pl.ds(seg_base, SEGE)], s_v)
        pltpu.sync_copy(dst_hbm.at[pl.ds(seg_base, SEGE)], d_v)
        pltpu.sync_copy(lab_hbm.at[pl.ds(seg_base, SEGE)], l_v)

        def body(b, off):
            sl = pl.ds(b * K, K)
            idst = d_v[sl]
            isrc = s_v[sl]
            zz = jnp.zeros((L,), jnp.int32)
            e = (plsc.load_gather(p_v, [zz, idst])
                 + plsc.load_gather(q_v, [zz, isrc]))
            e = jnp.where(e > 0, e, 0.2 * e)
            m = l_v[sl] > 0.5
            w = jnp.where(m, jnp.exp(e), 0.0)
            plsc.addupdate_scatter(den_v, [zz, idst], w)
            osl = pl.ds(off, K)
            plsc.store_compressed(cs_v.at[osl], isrc, mask=m)
            plsc.store_compressed(cd_v.at[osl], idst, mask=m)
            plsc.store_compressed(cw_v.at[osl], w, mask=m)
            cnt = plsc.all_reduce_population_count(m)[0]
            return off + cnt
        return lax.fori_loop(0, NBS, body, off)
    off = lax.fori_loop(0, NSEG, seg_body, 0)

    padto = lax.max(((off + R * K - 1) // (R * K)) * (R * K), R * K)
    zsrc = jnp.zeros((L,), jnp.int32)
    zdst = jnp.zeros((L,), jnp.int32) + N
    zw = jnp.zeros((L,), jnp.float32)
    for t in range(R):
        tsl = pl.ds(off + t * L, L)
        cs_v[tsl] = zsrc
        cd_v[tsl] = zdst
        cw_v[tsl] = zw
    cnt_v[...] = jnp.zeros((L,), jnp.int32) + padto

    pltpu.sync_copy(cs_v, cs_hbm.at[pl.ds(wid * EWP, EWP)])
    pltpu.sync_copy(cd_v, cd_hbm.at[pl.ds(wid * EWP, EWP)])
    pltpu.sync_copy(cw_v, cw_hbm.at[pl.ds(wid * EWP, EWP)])
    pltpu.sync_copy(cnt_v, cnt_hbm.at[pl.ds(wid * 16, 16)])
    pltpu.sync_copy(den_v, den_hbm.at[wid])


@functools.partial(
    pl.kernel,
    mesh=_mesh,
    compiler_params=pltpu.CompilerParams(needs_layout_passes=False),
    out_type=jax.ShapeDtypeStruct((NC, NPAD, D), jnp.float32),
    scratch_types=[
        pltpu.VMEM((SEG2,), jnp.int32),
        pltpu.VMEM((NBS2, K), jnp.int32),
        pltpu.VMEM((SEG2,), jnp.float32),
        pltpu.VMEM((16,), jnp.int32),
        pltpu.VMEM((R, K, D), jnp.float32),
        pltpu.VMEM_SHARED((NPAD, D), jnp.float32),
        [pltpu.SemaphoreType.DMA] * R,
        [pltpu.SemaphoreType.DMA] * R,
        [pltpu.SemaphoreType.DMA] * R,
    ],
)
def _edge_agg(cs_hbm, cd4_hbm, cw_hbm, cnt_hbm, zr_hbm, v_hbm,
              agg_hbm,
              s_v, d2_v, w_v, cnt_v, rows_v, agg_s, gsem, gsem2, ssem):
    cid = lax.axis_index("c")
    sid = lax.axis_index("s")
    wid = sid * NC + cid

    pltpu.sync_copy(cnt_hbm.at[pl.ds(wid * 16, 16)], cnt_v)
    nb = cnt_v[...][0] // K

    pltpu.sync_copy(zr_hbm, agg_s.at[pl.ds(sid * RPT, RPT)])
    plsc.subcore_barrier()

    H = K // 2

    def _gather_descA(b, slot):
        return pltpu.make_async_copy(
            v_hbm.at[s_v.at[pl.ds(b * K, H)]],
            rows_v.at[slot, pl.ds(0, H)], gsem[slot])

    def _gather_descB(b, slot):
        return pltpu.make_async_copy(
            v_hbm.at[s_v.at[pl.ds(b * K + H, H)]],
            rows_v.at[slot, pl.ds(H, H)], gsem2[slot])

    def _scatter_desc(b, slot):
        return pltpu.make_async_copy(
            rows_v.at[slot], agg_s.at[d2_v.at[b]], ssem[slot])

    def seg_body(g, _):
        pltpu.sync_copy(cs_hbm.at[pl.ds(wid * EWP + g * SEG2, SEG2)], s_v)
        pltpu.sync_copy(cd4_hbm.at[wid, g], d2_v)
        pltpu.sync_copy(cw_hbm.at[pl.ds(wid * EWP + g * SEG2, SEG2)], w_v)
        nbs_g = jnp.minimum(NBS2, nb - g * NBS2)

        for b0 in range(G):
            _gather_descA(b0, b0).start()
            _gather_descB(b0, b0).start()

        def body(grp, _):
            for slot in range(R):
                b = grp * R + slot
                @pl.when(b >= R - G)
                def _():
                    _scatter_desc(b - (R - G), (slot + G) % R).wait()

                @pl.when(b + G < nbs_g)
                def _():
                    _gather_descA(b + G, (slot + G) % R).start()
                    _gather_descB(b + G, (slot + G) % R).start()

                w = w_v[pl.ds(b * K, K)]
                _gather_descA(b, slot).wait()
                _gather_descB(b, slot).wait()
                for j in range(K):
                    wj = w[j]
                    for c in range(D // L):
                        cc = pl.ds(c * L, L)
                        rows_v[slot, j, cc] = rows_v[slot, j, cc] * wj
                _scatter_desc(b, slot).start(add=True)
            return 0
        lax.fori_loop(0, nbs_g // R, body, 0)

        for tail in range(R - G):
            b = nbs_g - (R - G) + tail
            _scatter_desc(b, G + tail).wait()
        return 0
    lax.fori_loop(0, (nb + NBS2 - 1) // NBS2, seg_body, 0)

    plsc.subcore_barrier()
    pltpu.sync_copy(agg_s.at[pl.ds(sid * RPT, RPT)],
                    agg_hbm.at[cid, pl.ds(sid * RPT, RPT)])


def _fin_body(x_ref, agg_ref, den_ref, o_ref):
    den = jnp.sum(den_ref[...], axis=(0, 1))[:, None]
    agg = (agg_ref[0] + agg_ref[1])[:N]
    safe = jnp.where(den > 0, den, 1.0)
    o_ref[...] = x_ref[...] + jnp.where(den > 0, agg / safe, 0.0)


_fin = pl.pallas_call(
    _fin_body,
    out_shape=jax.ShapeDtypeStruct((N, D), jnp.float32),
)


def kernel(event_embeddings, event_pairs, coreference_labels,
           W_V_w, W_V_b, W_R_w, a_w, a_b):
    src = event_pairs[:, 0]
    dst = event_pairs[:, 1]
    lab = coreference_labels[:, 0]
    ab = jnp.reshape(a_b, (1,))
    V, p, q = _dense(event_embeddings, W_V_w, W_R_w, W_V_b, a_w, ab)
    cs, cd, cw, cnt, den = _edge_w(p, q, src, dst, lab)
    cd4 = cd.reshape(NW, NSEG2, NBS2, K)
    zr = jnp.zeros((RPT, D), jnp.float32)
    agg = _edge_agg(cs, cd4, cw, cnt, zr, V)
    return _fin(event_embeddings, agg, den)


# speedup vs baseline: 1.0432x; 1.0432x over previous
import functools

import jax
import jax.numpy as jnp
from jax import lax
from jax.experimental import pallas as pl
from jax.experimental.pallas import tpu as pltpu
from jax.experimental.pallas import tpu_sc as plsc

N = 10000
D = 128
E = 320000
NC = 2
NS = 16
NW = NC * NS
EW = E // NW
L = 16
K = 16
SEGE = 2000
NSEG = EW // SEGE
NBS = SEGE // K
R = 8
G = 6
NBS2 = 128
SEG2 = NBS2 * K
NSEG2 = 6
EWP = NSEG2 * SEG2
NPAD = 10112
RPT = NPAD // NS


def _dense_body(x_ref, wv_ref, wr_ref, bv_ref, aw_ref, ab_ref,
                v_ref, p_ref, q_ref):
    x = x_ref[...]
    v_ref[...] = lax.dot_general(
        x, wv_ref[...], (((1,), (1,)), ((), ())),
        precision=lax.Precision.HIGHEST,
        preferred_element_type=jnp.float32) + bv_ref[...][None, :]
    a1 = aw_ref[0:D][None, :]
    a2 = aw_ref[D:2 * D][None, :]
    w2 = lax.dot_general(a2, wr_ref[...], (((1,), (0,)), ((), ())),
                         precision=lax.Precision.HIGHEST,
                         preferred_element_type=jnp.float32)
    p_ref[...] = lax.dot_general(a1, x, (((1,), (1,)), ((), ())),
                                 precision=lax.Precision.HIGHEST,
                                 preferred_element_type=jnp.float32) + ab_ref[0]
    q_ref[...] = lax.dot_general(w2, x, (((1,), (1,)), ((), ())),
                                 precision=lax.Precision.HIGHEST,
                                 preferred_element_type=jnp.float32)


_dense = pl.pallas_call(
    _dense_body,
    out_shape=(jax.ShapeDtypeStruct((N, D), jnp.float32),
               jax.ShapeDtypeStruct((1, N), jnp.float32),
               jax.ShapeDtypeStruct((1, N), jnp.float32)),
    in_specs=[pl.BlockSpec(memory_space=pltpu.VMEM)] * 5
             + [pl.BlockSpec(memory_space=pltpu.SMEM)],
)


_mesh = plsc.VectorSubcoreMesh(core_axis_name="c", subcore_axis_name="s")


@functools.partial(
    pl.kernel,
    mesh=_mesh,
    compiler_params=pltpu.CompilerParams(needs_layout_passes=False),
    out_type=(jax.ShapeDtypeStruct((NW * EWP,), jnp.int32),
              jax.ShapeDtypeStruct((NW * EWP,), jnp.int32),
              jax.ShapeDtypeStruct((NW * EWP,), jnp.float32),
              jax.ShapeDtypeStruct((NW * 16,), jnp.int32),
              jax.ShapeDtypeStruct((NW, 1, N), jnp.float32)),
    scratch_types=[
        pltpu.VMEM((1, N), jnp.float32),
        pltpu.VMEM((1, N), jnp.float32),
        pltpu.VMEM((SEGE,), jnp.int32),
        pltpu.VMEM((SEGE,), jnp.int32),
        pltpu.VMEM((SEGE,), jnp.float32),
        pltpu.VMEM((1, N), jnp.float32),
        pltpu.VMEM((EWP,), jnp.int32),
        pltpu.VMEM((EWP,), jnp.int32),
        pltpu.VMEM((EWP,), jnp.float32),
        pltpu.VMEM((16,), jnp.int32),
    ],
)
def _edge_w(p_hbm, q_hbm, src_hbm, dst_hbm, lab_hbm,
            cs_hbm, cd_hbm, cw_hbm, cnt_hbm, den_hbm,
            p_v, q_v, s_v, d_v, l_v, den_v, cs_v, cd_v, cw_v, cnt_v):
    cid = lax.axis_index("c")
    sid = lax.axis_index("s")
    wid = sid * NC + cid
    base = wid * EW

    pltpu.sync_copy(p_hbm, p_v)
    pltpu.sync_copy(q_hbm, q_v)

    def zero_body(i, _):
        den_v[0, pl.ds(i * L, L)] = jnp.zeros((L,), jnp.float32)
        return 0
    lax.fori_loop(0, N // L, zero_body, 0)

    def seg_body(g, off):
        seg_base = base + g * SEGE
        pltpu.sync_copy(src_hbm.at[pl.ds(seg_base, SEGE)], s_v)
        pltpu.sync_copy(dst_hbm.at[pl.ds(seg_base, SEGE)], d_v)
        pltpu.sync_copy(lab_hbm.at[pl.ds(seg_base, SEGE)], l_v)

        def body(b, off):
            sl = pl.ds(b * K, K)
            idst = d_v[sl]
            isrc = s_v[sl]
            zz = jnp.zeros((L,), jnp.int32)
            e = (plsc.load_gather(p_v, [zz, idst])
                 + plsc.load_gather(q_v, [zz, isrc]))
            e = jnp.where(e > 0, e, 0.2 * e)
            m = l_v[sl] > 0.5
            w = jnp.where(m, jnp.exp(e), 0.0)
            plsc.addupdate_scatter(den_v, [zz, idst], w)
            osl = pl.ds(off, K)
            plsc.store_compressed(cs_v.at[osl], isrc, mask=m)
            plsc.store_compressed(cd_v.at[osl], idst, mask=m)
            plsc.store_compressed(cw_v.at[osl], w, mask=m)
            cnt = plsc.all_reduce_population_count(m)[0]
            return off + cnt
        return lax.fori_loop(0, NBS, body, off)
    off = lax.fori_loop(0, NSEG, seg_body, 0)

    padto = lax.max(((off + R * K - 1) // (R * K)) * (R * K), R * K)
    zsrc = jnp.zeros((L,), jnp.int32)
    zdst = jnp.zeros((L,), jnp.int32) + N
    zw = jnp.zeros((L,), jnp.float32)
    for t in range(R):
        tsl = pl.ds(off + t * L, L)
        cs_v[tsl] = zsrc
        cd_v[tsl] = zdst
        cw_v[tsl] = zw
    cnt_v[...] = jnp.zeros((L,), jnp.int32) + padto

    pltpu.sync_copy(cs_v, cs_hbm.at[pl.ds(wid * EWP, EWP)])
    pltpu.sync_copy(cd_v, cd_hbm.at[pl.ds(wid * EWP, EWP)])
    pltpu.sync_copy(cw_v, cw_hbm.at[pl.ds(wid * EWP, EWP)])
    pltpu.sync_copy(cnt_v, cnt_hbm.at[pl.ds(wid * 16, 16)])
    pltpu.sync_copy(den_v, den_hbm.at[wid])


@functools.partial(
    pl.kernel,
    mesh=_mesh,
    compiler_params=pltpu.CompilerParams(needs_layout_passes=False),
    out_type=jax.ShapeDtypeStruct((NC, NPAD, D), jnp.float32),
    scratch_types=[
        pltpu.VMEM((SEG2,), jnp.int32),
        pltpu.VMEM((NBS2, K), jnp.int32),
        pltpu.VMEM((SEG2,), jnp.float32),
        pltpu.VMEM((16,), jnp.int32),
        pltpu.VMEM((R, K, D), jnp.float32),
        pltpu.VMEM_SHARED((NPAD, D), jnp.float32),
        [pltpu.SemaphoreType.DMA] * R,
        [pltpu.SemaphoreType.DMA] * R,
    ],
)
def _edge_agg(cs_hbm, cd4_hbm, cw_hbm, cnt_hbm, zr_hbm, v_hbm,
              agg_hbm,
              s_v, d2_v, w_v, cnt_v, rows_v, agg_s, gsem, ssem):
    cid = lax.axis_index("c")
    sid = lax.axis_index("s")
    wid = sid * NC + cid

    pltpu.sync_copy(cnt_hbm.at[pl.ds(wid * 16, 16)], cnt_v)
    nb = cnt_v[...][0] // K

    pltpu.sync_copy(zr_hbm, agg_s.at[pl.ds(sid * RPT, RPT)])
    plsc.subcore_barrier()

    def _gather_desc(b, slot):
        return pltpu.make_async_copy(
            v_hbm.at[s_v.at[pl.ds(b * K, K)]], rows_v.at[slot], gsem[slot])

    def _scatter_desc(b, slot):
        return pltpu.make_async_copy(
            rows_v.at[slot], agg_s.at[d2_v.at[b]], ssem[slot])

    def seg_body(g, _):
        pltpu.sync_copy(cs_hbm.at[pl.ds(wid * EWP + g * SEG2, SEG2)], s_v)
        pltpu.sync_copy(cd4_hbm.at[wid, g], d2_v)
        pltpu.sync_copy(cw_hbm.at[pl.ds(wid * EWP + g * SEG2, SEG2)], w_v)
        nbs_g = jnp.minimum(NBS2, nb - g * NBS2)

        for b0 in range(G):
            _gather_desc(b0, b0).start()

        def body(grp, _):
            for slot in range(R):
                b = grp * R + slot
                @pl.when(b >= R - G)
                def _():
                    _scatter_desc(b - (R - G), (slot + G) % R).wait()

                @pl.when(b + G < nbs_g)
                def _():
                    _gather_desc(b + G, (slot + G) % R).start()

                w = w_v[pl.ds(b * K, K)]
                _gather_desc(b, slot).wait()
                for j in range(K):
                    wj = w[j]
                    for c in range(D // L):
                        cc = pl.ds(c * L, L)
                        rows_v[slot, j, cc] = rows_v[slot, j, cc] * wj
                _scatter_desc(b, slot).start(add=True)
            return 0
        lax.fori_loop(0, nbs_g // R, body, 0)

        for tail in range(R - G):
            b = nbs_g - (R - G) + tail
            _scatter_desc(b, G + tail).wait()
        return 0
    lax.fori_loop(0, (nb + NBS2 - 1) // NBS2, seg_body, 0)

    plsc.subcore_barrier()
    pltpu.sync_copy(agg_s.at[pl.ds(sid * RPT, RPT)],
                    agg_hbm.at[cid, pl.ds(sid * RPT, RPT)])


def _fin_body(x_ref, agg_ref, den_ref, o_ref):
    den = jnp.sum(den_ref[...], axis=(0, 1))[:, None]
    agg = (agg_ref[0] + agg_ref[1])[:N]
    safe = jnp.where(den > 0, den, 1.0)
    o_ref[...] = x_ref[...] + jnp.where(den > 0, agg / safe, 0.0)


_fin = pl.pallas_call(
    _fin_body,
    out_shape=jax.ShapeDtypeStruct((N, D), jnp.float32),
)


def kernel(event_embeddings, event_pairs, coreference_labels,
           W_V_w, W_V_b, W_R_w, a_w, a_b):
    src = event_pairs[:, 0]
    dst = event_pairs[:, 1]
    lab = coreference_labels[:, 0]
    ab = jnp.reshape(a_b, (1,))
    V, p, q = _dense(event_embeddings, W_V_w, W_R_w, W_V_b, a_w, ab)
    cs, cd, cw, cnt, den = _edge_w(p, q, src, dst, lab)
    cd4 = cd.reshape(NW, NSEG2, NBS2, K)
    zr = jnp.zeros((RPT, D), jnp.float32)
    agg = _edge_agg(cs, cd4, cw, cnt, zr, V)
    return _fin(event_embeddings, agg, den)
